# baseline (speedup 1.0000x reference)
import jax
import jax.numpy as jnp
from jax import lax
from jax.experimental import pallas as pl
from jax.experimental.pallas import tpu as pltpu
from jax.experimental.pallas import tpu_sc as plsc

_N = 10000
_E = 320000
_D = 128
_NC = 2
_NS = 16
_NW = _NC * _NS
_CHUNK = 128
_TOTCH = _E // _CHUNK
_NCHUNK = -(-_TOTCH // _NW)
_ND = 10112
_WB = 632
_WBL = _N - (_NS - 1) * _WB

_mesh = plsc.VectorSubcoreMesh(core_axis_name="c", subcore_axis_name="s")


def _deg_body(dst_hbm, ones_hbm, init_hbm, out_hbm,
              dbuf, ones_v, acc):
    cid = lax.axis_index("c")
    sid = lax.axis_index("s")
    wid = cid * _NS + sid
    base = wid * _NCHUNK

    @pl.when(sid == 0)
    def _():
        pltpu.sync_copy(init_hbm.at[pl.ds(cid * _ND, _ND)], acc)

    pltpu.sync_copy(ones_hbm, ones_v)
    plsc.subcore_barrier()

    def chunk(j, carry):
        @pl.when(base + j < _TOTCH)
        def _():
            pltpu.sync_copy(
                dst_hbm.at[pl.ds((base + j) * _CHUNK, _CHUNK)], dbuf)
            pltpu.sync_copy(ones_v, acc.at[dbuf], add=True)
        return carry

    lax.fori_loop(0, _NCHUNK, chunk, 0)
    plsc.subcore_barrier()

    @pl.when(sid == 0)
    def _():
        pltpu.sync_copy(acc, out_hbm.at[cid, 0])


_deg_kernel = pl.kernel(
    _deg_body,
    out_type=jax.ShapeDtypeStruct((_NC, 1, _ND), jnp.float32),
    mesh=_mesh,
    compiler_params=pltpu.CompilerParams(skip_device_barrier=True),
    scratch_types=[
        pltpu.VMEM((_CHUNK,), jnp.int32),
        pltpu.VMEM((_CHUNK,), jnp.float32),
        pltpu.VMEM_SHARED((_ND,), jnp.float32),
    ],
)


def _agg_body(table_hbm, src_hbm, dst_hbm, zeros_hbm, out_hbm,
              sb0, sb1, sb2, db0, db1, db2, rows0, rows1, rows2,
              gs0, gs1, gs2, is0, is1, is2, acc):
    cid = lax.axis_index("c")
    sid = lax.axis_index("s")
    wid = cid * _NS + sid
    base = wid * _NCHUNK

    @pl.when(jnp.logical_and(sid == 0, cid == 0))
    def _():
        pltpu.sync_copy(table_hbm, acc)

    @pl.when(jnp.logical_and(sid == 0, cid == 1))
    def _():
        pltpu.sync_copy(zeros_hbm, acc)

    plsc.subcore_barrier()

    slots = ((sb0, db0, rows0, gs0, is0),
             (sb1, db1, rows1, gs1, is1),
             (sb2, db2, rows2, gs2, is2))

    def _idx(ref, j):
        return ref.at[pl.ds((base + j) * _CHUNK, _CHUNK)]

    for b, (sb, db, rr, gs, isem) in enumerate(slots):
        pltpu.sync_copy(_idx(src_hbm, b), sb)
        pltpu.sync_copy(_idx(dst_hbm, b), db)
        pltpu.async_copy(table_hbm.at[sb], rr, gs)

    def body(i, carry):
        for b, (sb, db, rr, gs, isem) in enumerate(slots):
            j = 3 * i + b

            @pl.when(jnp.logical_and(j < _NCHUNK, base + j < _TOTCH))
            def _():
                pltpu.make_async_copy(table_hbm.at[sb], rr, gs).wait()
                pltpu.sync_copy(rr, acc.at[db], add=True)

                @pl.when(jnp.logical_and(j + 3 < _NCHUNK,
                                         base + j + 3 < _TOTCH))
                def _():
                    pltpu.async_copy(_idx(src_hbm, j + 3), sb, isem)
                    pltpu.sync_copy(_idx(dst_hbm, j + 3), db)
                    pltpu.make_async_copy(
                        _idx(src_hbm, j + 3), sb, isem).wait()
                    pltpu.async_copy(table_hbm.at[sb], rr, gs)
        return carry

    lax.fori_loop(0, (_NCHUNK + 2) // 3, body, 0)
    plsc.subcore_barrier()

    @pl.when(sid < _NS - 1)
    def _():
        pltpu.sync_copy(acc.at[pl.ds(sid * _WB, _WB)],
                        out_hbm.at[cid, pl.ds(sid * _WB, _WB)])

    @pl.when(sid == _NS - 1)
    def _():
        pltpu.sync_copy(acc.at[pl.ds((_NS - 1) * _WB, _WBL)],
                        out_hbm.at[cid, pl.ds((_NS - 1) * _WB, _WBL)])


_agg_kernel = pl.kernel(
    _agg_body,
    out_type=jax.ShapeDtypeStruct((_NC, _N, _D), jnp.float32),
    mesh=_mesh,
    compiler_params=pltpu.CompilerParams(skip_device_barrier=True),
    scratch_types=(
        [pltpu.VMEM((_CHUNK,), jnp.int32)] * 6
        + [pltpu.VMEM((_CHUNK, _D), jnp.float32)] * 3
        + [pltpu.SemaphoreType.DMA] * 6
        + [pltpu.VMEM_SHARED((_N, _D), jnp.float32)]
    ),
)


_BLK = 1000


def _prep_body(d0_ref, d1_ref, x_ref, xs_ref, dinv_ref):
    deg = d0_ref[...] + d1_ref[...]
    dinv = lax.rsqrt(deg)
    dinv_ref[...] = dinv
    xs_ref[...] = x_ref[...] * dinv


def _mm_body(p0_ref, p1_ref, dinv_ref, w1_ref, b1_ref, w2_ref, out_ref):
    t = p0_ref[...] + p1_ref[...]
    dinv = dinv_ref[...]
    a = jnp.dot(t, w1_ref[...], preferred_element_type=jnp.float32)
    h = jnp.maximum(a * dinv + b1_ref[...], 0.0)
    g = jnp.dot(h, w2_ref[...], preferred_element_type=jnp.float32)
    out_ref[...] = g * dinv


def _fin_body(q0_ref, q1_ref, dinv_ref, b2_ref, out_ref):
    out_ref[...] = (q0_ref[...] + q1_ref[...]) * dinv_ref[...] + b2_ref[...]


def _row_spec(cols):
    return pl.BlockSpec((_BLK, cols), lambda i: (i, 0))


def _full_spec(r, c):
    return pl.BlockSpec((r, c), lambda i: (0, 0))


_prep_call = pl.pallas_call(
    _prep_body,
    grid=(_N // _BLK,),
    in_specs=[_row_spec(1), _row_spec(1), _row_spec(_D)],
    out_specs=[_row_spec(_D), _row_spec(1)],
    out_shape=[
        jax.ShapeDtypeStruct((_N, _D), jnp.float32),
        jax.ShapeDtypeStruct((_N, 1), jnp.float32),
    ],
)

_mm_call = pl.pallas_call(
    _mm_body,
    grid=(_N // _BLK,),
    in_specs=[
        _row_spec(_D), _row_spec(_D), _row_spec(1),
        _full_spec(128, 256), _full_spec(1, 256), _full_spec(256, 128),
    ],
    out_specs=_row_spec(_D),
    out_shape=jax.ShapeDtypeStruct((_N, _D), jnp.float32),
)

_fin_call = pl.pallas_call(
    _fin_body,
    grid=(_N // _BLK,),
    in_specs=[_row_spec(_D), _row_spec(_D), _row_spec(1), _full_spec(1, _D)],
    out_specs=_row_spec(_D),
    out_shape=jax.ShapeDtypeStruct((_N, _D), jnp.float32),
)


def kernel(x, edge_index, W1, b1, W2, b2):
    ei = edge_index.astype(jnp.int32)
    src = ei[0]
    dst = ei[1]

    zeros_nd = jnp.zeros((_N, _D), jnp.float32)
    deg_init = jnp.concatenate(
        [jnp.ones((_ND,), jnp.float32), jnp.zeros((_ND,), jnp.float32)])
    ones_c = jnp.ones((_CHUNK,), jnp.float32)

    degp = _deg_kernel(dst, ones_c, deg_init)
    d0 = degp[0, 0, :_N].reshape(_N, 1)
    d1 = degp[1, 0, :_N].reshape(_N, 1)
    xs, dinv = _prep_call(d0, d1, x)

    p = _agg_kernel(xs, src, dst, zeros_nd)
    g2 = _mm_call(p[0], p[1], dinv, W1, b1.reshape(1, -1), W2)

    q = _agg_kernel(g2, src, dst, zeros_nd)
    out = _fin_call(q[0], q[1], dinv, b2.reshape(1, -1))
    return out


# speedup vs baseline: 1.0159x; 1.0159x over previous
import jax
import jax.numpy as jnp
from jax import lax
from jax.experimental import pallas as pl
from jax.experimental.pallas import tpu as pltpu
from jax.experimental.pallas import tpu_sc as plsc

_N = 10000
_E = 320000
_D = 128
_NC = 2
_NS = 16
_NW = _NC * _NS
_CHUNK = 128
_TOTCH = _E // _CHUNK
_NCHUNK = -(-_TOTCH // _NW)
_ND = 10112
_WB = 632
_WBL = _N - (_NS - 1) * _WB

_mesh = plsc.VectorSubcoreMesh(core_axis_name="c", subcore_axis_name="s")


def _deg_body(dst_hbm, ones_hbm, init_hbm, out_hbm,
              dbuf, ones_v, acc):
    cid = lax.axis_index("c")
    sid = lax.axis_index("s")
    wid = cid * _NS + sid
    base = wid * _NCHUNK

    @pl.when(sid == 0)
    def _():
        pltpu.sync_copy(init_hbm.at[pl.ds(cid * _ND, _ND)], acc)

    pltpu.sync_copy(ones_hbm, ones_v)
    plsc.subcore_barrier()

    def chunk(j, carry):
        @pl.when(base + j < _TOTCH)
        def _():
            pltpu.sync_copy(
                dst_hbm.at[pl.ds((base + j) * _CHUNK, _CHUNK)], dbuf)
            pltpu.sync_copy(ones_v, acc.at[dbuf], add=True)
        return carry

    lax.fori_loop(0, _NCHUNK, chunk, 0)
    plsc.subcore_barrier()

    @pl.when(sid == 0)
    def _():
        pltpu.sync_copy(acc, out_hbm.at[cid, 0])


_deg_kernel = pl.kernel(
    _deg_body,
    out_type=jax.ShapeDtypeStruct((_NC, 1, _ND), jnp.float32),
    mesh=_mesh,
    scratch_types=[
        pltpu.VMEM((_CHUNK,), jnp.int32),
        pltpu.VMEM((_CHUNK,), jnp.float32),
        pltpu.VMEM_SHARED((_ND,), jnp.float32),
    ],
)


def _agg_body(table_hbm, src_hbm, dst_hbm, zeros_hbm, out_hbm,
              sb0, sb1, sb2, db0, db1, db2, rows0, rows1, rows2,
              gs0, gs1, gs2, is0, is1, is2, acc):
    cid = lax.axis_index("c")
    sid = lax.axis_index("s")
    wid = cid * _NS + sid
    base = wid * _NCHUNK

    @pl.when(jnp.logical_and(sid == 0, cid == 0))
    def _():
        pltpu.sync_copy(table_hbm, acc)

    @pl.when(jnp.logical_and(sid == 0, cid == 1))
    def _():
        pltpu.sync_copy(zeros_hbm, acc)

    plsc.subcore_barrier()

    slots = ((sb0, db0, rows0, gs0, is0),
             (sb1, db1, rows1, gs1, is1),
             (sb2, db2, rows2, gs2, is2))

    def _idx(ref, j):
        return ref.at[pl.ds((base + j) * _CHUNK, _CHUNK)]

    for b, (sb, db, rr, gs, isem) in enumerate(slots):
        pltpu.sync_copy(_idx(src_hbm, b), sb)
        pltpu.sync_copy(_idx(dst_hbm, b), db)
        pltpu.async_copy(table_hbm.at[sb], rr, gs)

    def body(i, carry):
        for b, (sb, db, rr, gs, isem) in enumerate(slots):
            j = 3 * i + b

            @pl.when(jnp.logical_and(j < _NCHUNK, base + j < _TOTCH))
            def _():
                pltpu.make_async_copy(table_hbm.at[sb], rr, gs).wait()
                pltpu.sync_copy(rr, acc.at[db], add=True)

                @pl.when(jnp.logical_and(j + 3 < _NCHUNK,
                                         base + j + 3 < _TOTCH))
                def _():
                    pltpu.async_copy(_idx(src_hbm, j + 3), sb, isem)
                    pltpu.sync_copy(_idx(dst_hbm, j + 3), db)
                    pltpu.make_async_copy(
                        _idx(src_hbm, j + 3), sb, isem).wait()
                    pltpu.async_copy(table_hbm.at[sb], rr, gs)
        return carry

    lax.fori_loop(0, (_NCHUNK + 2) // 3, body, 0)
    plsc.subcore_barrier()

    @pl.when(sid < _NS - 1)
    def _():
        pltpu.sync_copy(acc.at[pl.ds(sid * _WB, _WB)],
                        out_hbm.at[cid, pl.ds(sid * _WB, _WB)])

    @pl.when(sid == _NS - 1)
    def _():
        pltpu.sync_copy(acc.at[pl.ds((_NS - 1) * _WB, _WBL)],
                        out_hbm.at[cid, pl.ds((_NS - 1) * _WB, _WBL)])


_agg_kernel = pl.kernel(
    _agg_body,
    out_type=jax.ShapeDtypeStruct((_NC, _N, _D), jnp.float32),
    mesh=_mesh,
    scratch_types=(
        [pltpu.VMEM((_CHUNK,), jnp.int32)] * 6
        + [pltpu.VMEM((_CHUNK, _D), jnp.float32)] * 3
        + [pltpu.SemaphoreType.DMA] * 6
        + [pltpu.VMEM_SHARED((_N, _D), jnp.float32)]
    ),
)


_BLK = 2000


def _prep_body(d0_ref, d1_ref, x_ref, xs_ref, dinv_ref):
    deg = d0_ref[...] + d1_ref[...]
    dinv = lax.rsqrt(deg)
    dinv_ref[...] = dinv
    xs_ref[...] = x_ref[...] * dinv


def _mm_body(p0_ref, p1_ref, dinv_ref, w1_ref, b1_ref, w2_ref, out_ref):
    t = p0_ref[...] + p1_ref[...]
    dinv = dinv_ref[...]
    a = jnp.dot(t, w1_ref[...], preferred_element_type=jnp.float32)
    h = jnp.maximum(a * dinv + b1_ref[...], 0.0)
    g = jnp.dot(h, w2_ref[...], preferred_element_type=jnp.float32)
    out_ref[...] = g * dinv


def _fin_body(q0_ref, q1_ref, dinv_ref, b2_ref, out_ref):
    out_ref[...] = (q0_ref[...] + q1_ref[...]) * dinv_ref[...] + b2_ref[...]


def _row_spec(cols):
    return pl.BlockSpec((_BLK, cols), lambda i: (i, 0))


def _full_spec(r, c):
    return pl.BlockSpec((r, c), lambda i: (0, 0))


_prep_call = pl.pallas_call(
    _prep_body,
    grid=(_N // _BLK,),
    in_specs=[_row_spec(1), _row_spec(1), _row_spec(_D)],
    out_specs=[_row_spec(_D), _row_spec(1)],
    out_shape=[
        jax.ShapeDtypeStruct((_N, _D), jnp.float32),
        jax.ShapeDtypeStruct((_N, 1), jnp.float32),
    ],
)

_mm_call = pl.pallas_call(
    _mm_body,
    grid=(_N // _BLK,),
    in_specs=[
        _row_spec(_D), _row_spec(_D), _row_spec(1),
        _full_spec(128, 256), _full_spec(1, 256), _full_spec(256, 128),
    ],
    out_specs=_row_spec(_D),
    out_shape=jax.ShapeDtypeStruct((_N, _D), jnp.float32),
)

_fin_call = pl.pallas_call(
    _fin_body,
    grid=(_N // _BLK,),
    in_specs=[_row_spec(_D), _row_spec(_D), _row_spec(1), _full_spec(1, _D)],
    out_specs=_row_spec(_D),
    out_shape=jax.ShapeDtypeStruct((_N, _D), jnp.float32),
)


def kernel(x, edge_index, W1, b1, W2, b2):
    ei = edge_index.astype(jnp.int32)
    src = ei[0]
    dst = ei[1]

    zeros_nd = jnp.zeros((_N, _D), jnp.float32)
    deg_init = jnp.concatenate(
        [jnp.ones((_ND,), jnp.float32), jnp.zeros((_ND,), jnp.float32)])
    ones_c = jnp.ones((_CHUNK,), jnp.float32)

    degp = _deg_kernel(dst, ones_c, deg_init)
    d0 = degp[0, 0, :_N].reshape(_N, 1)
    d1 = degp[1, 0, :_N].reshape(_N, 1)
    xs, dinv = _prep_call(d0, d1, x)

    p = _agg_kernel(xs, src, dst, zeros_nd)
    g2 = _mm_call(p[0], p[1], dinv, W1, b1.reshape(1, -1), W2)

    q = _agg_kernel(g2, src, dst, zeros_nd)
    out = _fin_call(q[0], q[1], dinv, b2.reshape(1, -1))
    return out


# speedup vs baseline: 1.0801x; 1.0632x over previous
import jax
import jax.numpy as jnp
from jax import lax
from jax.experimental import pallas as pl
from jax.experimental.pallas import tpu as pltpu
from jax.experimental.pallas import tpu_sc as plsc

_N = 10000
_E = 320000
_D = 128
_NC = 2
_NS = 16
_NW = _NC * _NS
_CHUNK = 128
_TOTCH = _E // _CHUNK
_NCHUNK = -(-_TOTCH // _NW)
_ND = 10112
_WB = 632
_WBL = _N - (_NS - 1) * _WB

_mesh = plsc.VectorSubcoreMesh(core_axis_name="c", subcore_axis_name="s")


def _deg_body(dst_hbm, ones_hbm, init_hbm, out_hbm,
              dbuf0, dbuf1, ones_v, isem0, isem1, acc):
    cid = lax.axis_index("c")
    sid = lax.axis_index("s")
    wid = cid * _NS + sid
    base = wid * _NCHUNK

    @pl.when(sid == 0)
    def _():
        pltpu.sync_copy(init_hbm.at[pl.ds(cid * _ND, _ND)], acc)

    pltpu.sync_copy(ones_hbm, ones_v)
    plsc.subcore_barrier()

    def _dslice(j):
        return dst_hbm.at[pl.ds((base + j) * _CHUNK, _CHUNK)]

    dslots = ((dbuf0, isem0), (dbuf1, isem1))
    for b, (db, isem) in enumerate(dslots):
        pltpu.async_copy(_dslice(b), db, isem)

    def chunk(i, carry):
        for b, (db, isem) in enumerate(dslots):
            j = 2 * i + b

            @pl.when(jnp.logical_and(j < _NCHUNK, base + j < _TOTCH))
            def _():
                pltpu.make_async_copy(_dslice(j), db, isem).wait()
                pltpu.sync_copy(ones_v, acc.at[db], add=True)

                @pl.when(jnp.logical_and(j + 2 < _NCHUNK,
                                         base + j + 2 < _TOTCH))
                def _():
                    pltpu.async_copy(_dslice(j + 2), db, isem)
        return carry

    lax.fori_loop(0, (_NCHUNK + 1) // 2, chunk, 0)
    plsc.subcore_barrier()

    @pl.when(sid == 0)
    def _():
        pltpu.sync_copy(acc, out_hbm.at[cid, 0])


_deg_kernel = pl.kernel(
    _deg_body,
    out_type=jax.ShapeDtypeStruct((_NC, 1, _ND), jnp.float32),
    mesh=_mesh,
    scratch_types=[
        pltpu.VMEM((_CHUNK,), jnp.int32),
        pltpu.VMEM((_CHUNK,), jnp.int32),
        pltpu.VMEM((_CHUNK,), jnp.float32),
        pltpu.SemaphoreType.DMA,
        pltpu.SemaphoreType.DMA,
        pltpu.VMEM_SHARED((_ND,), jnp.float32),
    ],
)


def _agg_body(table_hbm, src_hbm, dst_hbm, zeros_hbm, out_hbm,
              sb0, sb1, sb2, db0, db1, db2, rows0, rows1, rows2,
              gs0, gs1, gs2, is0, is1, is2, acc):
    cid = lax.axis_index("c")
    sid = lax.axis_index("s")
    wid = cid * _NS + sid
    base = wid * _NCHUNK

    @pl.when(jnp.logical_and(sid == 0, cid == 0))
    def _():
        pltpu.sync_copy(table_hbm, acc)

    @pl.when(jnp.logical_and(sid == 0, cid == 1))
    def _():
        pltpu.sync_copy(zeros_hbm, acc)

    plsc.subcore_barrier()

    slots = ((sb0, db0, rows0, gs0, is0),
             (sb1, db1, rows1, gs1, is1),
             (sb2, db2, rows2, gs2, is2))

    def _idx(ref, j):
        return ref.at[pl.ds((base + j) * _CHUNK, _CHUNK)]

    for b, (sb, db, rr, gs, isem) in enumerate(slots):
        pltpu.sync_copy(_idx(src_hbm, b), sb)
        pltpu.sync_copy(_idx(dst_hbm, b), db)
        pltpu.async_copy(table_hbm.at[sb], rr, gs)

    def body(i, carry):
        for b, (sb, db, rr, gs, isem) in enumerate(slots):
            j = 3 * i + b

            @pl.when(jnp.logical_and(j < _NCHUNK, base + j < _TOTCH))
            def _():
                pltpu.make_async_copy(table_hbm.at[sb], rr, gs).wait()
                pltpu.sync_copy(rr, acc.at[db], add=True)

                @pl.when(jnp.logical_and(j + 3 < _NCHUNK,
                                         base + j + 3 < _TOTCH))
                def _():
                    pltpu.async_copy(_idx(src_hbm, j + 3), sb, isem)
                    pltpu.sync_copy(_idx(dst_hbm, j + 3), db)
                    pltpu.make_async_copy(
                        _idx(src_hbm, j + 3), sb, isem).wait()
                    pltpu.async_copy(table_hbm.at[sb], rr, gs)
        return carry

    lax.fori_loop(0, (_NCHUNK + 2) // 3, body, 0)
    plsc.subcore_barrier()

    @pl.when(sid < _NS - 1)
    def _():
        pltpu.sync_copy(acc.at[pl.ds(sid * _WB, _WB)],
                        out_hbm.at[cid, pl.ds(sid * _WB, _WB)])

    @pl.when(sid == _NS - 1)
    def _():
        pltpu.sync_copy(acc.at[pl.ds((_NS - 1) * _WB, _WBL)],
                        out_hbm.at[cid, pl.ds((_NS - 1) * _WB, _WBL)])


_agg_kernel = pl.kernel(
    _agg_body,
    out_type=jax.ShapeDtypeStruct((_NC, _N, _D), jnp.float32),
    mesh=_mesh,
    scratch_types=(
        [pltpu.VMEM((_CHUNK,), jnp.int32)] * 6
        + [pltpu.VMEM((_CHUNK, _D), jnp.float32)] * 3
        + [pltpu.SemaphoreType.DMA] * 6
        + [pltpu.VMEM_SHARED((_N, _D), jnp.float32)]
    ),
)


_BLK = 2000


def _prep_body(d0_ref, d1_ref, x_ref, xs_ref, dinv_ref):
    deg = d0_ref[...] + d1_ref[...]
    dinv = lax.rsqrt(deg)
    dinv_ref[...] = dinv
    xs_ref[...] = x_ref[...] * dinv


def _mm_body(p0_ref, p1_ref, dinv_ref, w1_ref, b1_ref, w2_ref, out_ref):
    t = p0_ref[...] + p1_ref[...]
    dinv = dinv_ref[...]
    a = jnp.dot(t, w1_ref[...], preferred_element_type=jnp.float32)
    h = jnp.maximum(a * dinv + b1_ref[...], 0.0)
    g = jnp.dot(h, w2_ref[...], preferred_element_type=jnp.float32)
    out_ref[...] = g * dinv


def _fin_body(q0_ref, q1_ref, dinv_ref, b2_ref, out_ref):
    out_ref[...] = (q0_ref[...] + q1_ref[...]) * dinv_ref[...] + b2_ref[...]


def _row_spec(cols):
    return pl.BlockSpec((_BLK, cols), lambda i: (i, 0))


def _full_spec(r, c):
    return pl.BlockSpec((r, c), lambda i: (0, 0))


_prep_call = pl.pallas_call(
    _prep_body,
    grid=(_N // _BLK,),
    in_specs=[_row_spec(1), _row_spec(1), _row_spec(_D)],
    out_specs=[_row_spec(_D), _row_spec(1)],
    out_shape=[
        jax.ShapeDtypeStruct((_N, _D), jnp.float32),
        jax.ShapeDtypeStruct((_N, 1), jnp.float32),
    ],
)

_mm_call = pl.pallas_call(
    _mm_body,
    grid=(_N // _BLK,),
    in_specs=[
        _row_spec(_D), _row_spec(_D), _row_spec(1),
        _full_spec(128, 256), _full_spec(1, 256), _full_spec(256, 128),
    ],
    out_specs=_row_spec(_D),
    out_shape=jax.ShapeDtypeStruct((_N, _D), jnp.float32),
)

_fin_call = pl.pallas_call(
    _fin_body,
    grid=(_N // _BLK,),
    in_specs=[_row_spec(_D), _row_spec(_D), _row_spec(1), _full_spec(1, _D)],
    out_specs=_row_spec(_D),
    out_shape=jax.ShapeDtypeStruct((_N, _D), jnp.float32),
)


def kernel(x, edge_index, W1, b1, W2, b2):
    ei = edge_index.astype(jnp.int32)
    src = ei[0]
    dst = ei[1]

    zeros_nd = jnp.zeros((_N, _D), jnp.float32)
    deg_init = jnp.concatenate(
        [jnp.ones((_ND,), jnp.float32), jnp.zeros((_ND,), jnp.float32)])
    ones_c = jnp.ones((_CHUNK,), jnp.float32)

    degp = _deg_kernel(dst, ones_c, deg_init)
    d0 = degp[0, 0, :_N].reshape(_N, 1)
    d1 = degp[1, 0, :_N].reshape(_N, 1)
    xs, dinv = _prep_call(d0, d1, x)

    p = _agg_kernel(xs, src, dst, zeros_nd)
    g2 = _mm_call(p[0], p[1], dinv, W1, b1.reshape(1, -1), W2)

    q = _agg_kernel(g2, src, dst, zeros_nd)
    out = _fin_call(q[0], q[1], dinv, b2.reshape(1, -1))
    return out


# speedup vs baseline: 1.2354x; 1.1438x over previous
import jax
import jax.numpy as jnp
from jax import lax
from jax.experimental import pallas as pl
from jax.experimental.pallas import tpu as pltpu
from jax.experimental.pallas import tpu_sc as plsc

_N = 10000
_E = 320000
_D = 128
_NC = 2
_NS = 16
_NW = _NC * _NS
_CHUNK = 128
_TOTCH = _E // _CHUNK
_NCHUNK = -(-_TOTCH // _NW)
_ND = 10112
_WB = 632
_WBL = _N - (_NS - 1) * _WB

_mesh = plsc.VectorSubcoreMesh(core_axis_name="c", subcore_axis_name="s")


def _deg_body(dst_hbm, ones_hbm, init_hbm, out_hbm,
              dbuf0, dbuf1, ones_v, isem0, isem1, acc):
    cid = lax.axis_index("c")
    sid = lax.axis_index("s")
    wid = cid * _NS + sid
    base = wid * _NCHUNK

    @pl.when(sid == 0)
    def _():
        pltpu.sync_copy(init_hbm.at[pl.ds(cid * _ND, _ND)], acc)

    pltpu.sync_copy(ones_hbm, ones_v)
    plsc.subcore_barrier()

    def _dslice(j):
        return dst_hbm.at[pl.ds((base + j) * _CHUNK, _CHUNK)]

    dslots = ((dbuf0, isem0), (dbuf1, isem1))
    for b, (db, isem) in enumerate(dslots):
        pltpu.async_copy(_dslice(b), db, isem)

    def chunk(i, carry):
        for b, (db, isem) in enumerate(dslots):
            j = 2 * i + b

            @pl.when(jnp.logical_and(j < _NCHUNK, base + j < _TOTCH))
            def _():
                pltpu.make_async_copy(_dslice(j), db, isem).wait()
                pltpu.sync_copy(ones_v, acc.at[db], add=True)

                @pl.when(jnp.logical_and(j + 2 < _NCHUNK,
                                         base + j + 2 < _TOTCH))
                def _():
                    pltpu.async_copy(_dslice(j + 2), db, isem)
        return carry

    lax.fori_loop(0, (_NCHUNK + 1) // 2, chunk, 0)
    plsc.subcore_barrier()

    @pl.when(sid == 0)
    def _():
        pltpu.sync_copy(acc, out_hbm.at[cid, 0])


_deg_kernel = pl.kernel(
    _deg_body,
    out_type=jax.ShapeDtypeStruct((_NC, 1, _ND), jnp.float32),
    mesh=_mesh,
    scratch_types=[
        pltpu.VMEM((_CHUNK,), jnp.int32),
        pltpu.VMEM((_CHUNK,), jnp.int32),
        pltpu.VMEM((_CHUNK,), jnp.float32),
        pltpu.SemaphoreType.DMA,
        pltpu.SemaphoreType.DMA,
        pltpu.VMEM_SHARED((_ND,), jnp.float32),
    ],
)


def _agg_body(table_hbm, src_hbm, dst_hbm, zeros_hbm, out_hbm,
              sb0, sb1, sb2, sb3, sb4, sb5, db0, db1, db2, db3, db4, db5,
              rows0, rows1, rows2, gs0, gs1, gs2,
              ss0, ss1, ss2, ss3, ss4, ss5, ds0, ds1, ds2, ds3, ds4, ds5,
              acc):
    cid = lax.axis_index("c")
    sid = lax.axis_index("s")
    wid = cid * _NS + sid
    base = wid * _NCHUNK

    @pl.when(jnp.logical_and(sid == 0, cid == 0))
    def _():
        pltpu.sync_copy(table_hbm, acc)

    @pl.when(jnp.logical_and(sid == 0, cid == 1))
    def _():
        pltpu.sync_copy(zeros_hbm, acc)

    plsc.subcore_barrier()

    sbs = (sb0, sb1, sb2, sb3, sb4, sb5)
    dbs = (db0, db1, db2, db3, db4, db5)
    rows = (rows0, rows1, rows2)
    gss = (gs0, gs1, gs2)
    sss = (ss0, ss1, ss2, ss3, ss4, ss5)
    dss = (ds0, ds1, ds2, ds3, ds4, ds5)

    def _idx(ref, j):
        return ref.at[pl.ds((base + j) * _CHUNK, _CHUNK)]

    def _valid(j):
        return jnp.logical_and(j < _NCHUNK, base + j < _TOTCH)

    for j in range(3):
        pltpu.sync_copy(_idx(src_hbm, j), sbs[j])
        pltpu.sync_copy(_idx(dst_hbm, j), dbs[j])
        pltpu.async_copy(table_hbm.at[sbs[j]], rows[j], gss[j])
    for j in range(3, 6):
        pltpu.async_copy(_idx(src_hbm, j), sbs[j], sss[j])
        pltpu.async_copy(_idx(dst_hbm, j), dbs[j], dss[j])

    def body(i, carry):
        for b in range(6):
            j = 6 * i + b
            s, s3, r = b, (b + 3) % 6, b % 3

            @pl.when(_valid(j))
            def _():
                pltpu.make_async_copy(
                    table_hbm.at[sbs[s]], rows[r], gss[r]).wait()
                pltpu.sync_copy(rows[r], acc.at[dbs[s]], add=True)

                @pl.when(_valid(j + 3))
                def _():
                    pltpu.make_async_copy(
                        _idx(src_hbm, j + 3), sbs[s3], sss[s3]).wait()
                    pltpu.make_async_copy(
                        _idx(dst_hbm, j + 3), dbs[s3], dss[s3]).wait()
                    pltpu.async_copy(
                        table_hbm.at[sbs[s3]], rows[r], gss[r])

                @pl.when(_valid(j + 6))
                def _():
                    pltpu.async_copy(_idx(src_hbm, j + 6), sbs[s], sss[s])
                    pltpu.async_copy(_idx(dst_hbm, j + 6), dbs[s], dss[s])
        return carry

    lax.fori_loop(0, (_NCHUNK + 5) // 6, body, 0)
    plsc.subcore_barrier()

    @pl.when(sid < _NS - 1)
    def _():
        pltpu.sync_copy(acc.at[pl.ds(sid * _WB, _WB)],
                        out_hbm.at[cid, pl.ds(sid * _WB, _WB)])

    @pl.when(sid == _NS - 1)
    def _():
        pltpu.sync_copy(acc.at[pl.ds((_NS - 1) * _WB, _WBL)],
                        out_hbm.at[cid, pl.ds((_NS - 1) * _WB, _WBL)])


_agg_kernel = pl.kernel(
    _agg_body,
    out_type=jax.ShapeDtypeStruct((_NC, _N, _D), jnp.float32),
    mesh=_mesh,
    scratch_types=(
        [pltpu.VMEM((_CHUNK,), jnp.int32)] * 12
        + [pltpu.VMEM((_CHUNK, _D), jnp.float32)] * 3
        + [pltpu.SemaphoreType.DMA] * 15
        + [pltpu.VMEM_SHARED((_N, _D), jnp.float32)]
    ),
)


_BLK = 2000


def _prep_body(d0_ref, d1_ref, x_ref, xs_ref, dinv_ref):
    deg = d0_ref[...] + d1_ref[...]
    dinv = lax.rsqrt(deg)
    dinv_ref[...] = dinv
    xs_ref[...] = x_ref[...] * dinv


def _mm_body(p0_ref, p1_ref, dinv_ref, w1_ref, b1_ref, w2_ref, out_ref):
    t = p0_ref[...] + p1_ref[...]
    dinv = dinv_ref[...]
    a = jnp.dot(t, w1_ref[...], preferred_element_type=jnp.float32)
    h = jnp.maximum(a * dinv + b1_ref[...], 0.0)
    g = jnp.dot(h, w2_ref[...], preferred_element_type=jnp.float32)
    out_ref[...] = g * dinv


def _fin_body(q0_ref, q1_ref, dinv_ref, b2_ref, out_ref):
    out_ref[...] = (q0_ref[...] + q1_ref[...]) * dinv_ref[...] + b2_ref[...]


def _row_spec(cols):
    return pl.BlockSpec((_BLK, cols), lambda i: (i, 0))


def _full_spec(r, c):
    return pl.BlockSpec((r, c), lambda i: (0, 0))


_prep_call = pl.pallas_call(
    _prep_body,
    grid=(_N // _BLK,),
    in_specs=[_row_spec(1), _row_spec(1), _row_spec(_D)],
    out_specs=[_row_spec(_D), _row_spec(1)],
    out_shape=[
        jax.ShapeDtypeStruct((_N, _D), jnp.float32),
        jax.ShapeDtypeStruct((_N, 1), jnp.float32),
    ],
)

_mm_call = pl.pallas_call(
    _mm_body,
    grid=(_N // _BLK,),
    in_specs=[
        _row_spec(_D), _row_spec(_D), _row_spec(1),
        _full_spec(128, 256), _full_spec(1, 256), _full_spec(256, 128),
    ],
    out_specs=_row_spec(_D),
    out_shape=jax.ShapeDtypeStruct((_N, _D), jnp.float32),
)

_fin_call = pl.pallas_call(
    _fin_body,
    grid=(_N // _BLK,),
    in_specs=[_row_spec(_D), _row_spec(_D), _row_spec(1), _full_spec(1, _D)],
    out_specs=_row_spec(_D),
    out_shape=jax.ShapeDtypeStruct((_N, _D), jnp.float32),
)


def kernel(x, edge_index, W1, b1, W2, b2):
    ei = edge_index.astype(jnp.int32)
    src = ei[0]
    dst = ei[1]

    zeros_nd = jnp.zeros((_N, _D), jnp.float32)
    deg_init = jnp.concatenate(
        [jnp.ones((_ND,), jnp.float32), jnp.zeros((_ND,), jnp.float32)])
    ones_c = jnp.ones((_CHUNK,), jnp.float32)

    degp = _deg_kernel(dst, ones_c, deg_init)
    d0 = degp[0, 0, :_N].reshape(_N, 1)
    d1 = degp[1, 0, :_N].reshape(_N, 1)
    xs, dinv = _prep_call(d0, d1, x)

    p = _agg_kernel(xs, src, dst, zeros_nd)
    g2 = _mm_call(p[0], p[1], dinv, W1, b1.reshape(1, -1), W2)

    q = _agg_kernel(g2, src, dst, zeros_nd)
    out = _fin_call(q[0], q[1], dinv, b2.reshape(1, -1))
    return out
